# Initial kernel scaffold; baseline (speedup 1.0000x reference)
import jax
import jax.numpy as jnp
from jax.experimental import pallas as pl


def kernel(x, edge_index, pos, Wc, bc, W0, b0, W1, b1):
    raise NotImplementedError("write your pallas kernel here")


# speedup vs baseline: 5.7127x; 5.7127x over previous
import functools

import jax
import jax.numpy as jnp
from jax import lax
from jax.experimental import pallas as pl
from jax.experimental.pallas import tpu as pltpu
from jax.experimental.pallas import tpu_sc as plsc

NC, NS, LANES = 2, 16, 16

N = 10000
E = 160000
F = 256
M = 4096

NP = 10240

_mesh = plsc.VectorSubcoreMesh(core_axis_name="c", subcore_axis_name="s")


@functools.partial(
    pl.kernel,
    out_type=jax.ShapeDtypeStruct((2 * NP,), jnp.float32),
    mesh=_mesh,
    scratch_types=[
        pltpu.VMEM((128,), jnp.int32),
        pltpu.VMEM((128,), jnp.int32),
        pltpu.VMEM((128,), jnp.float32),
        pltpu.VMEM((640,), jnp.float32),
    ],
)
def _mask_kernel(posflat_hbm, mask_hbm, idx_v, fidx_v, ones_v, zeros_v):
    c = lax.axis_index("c")
    s = lax.axis_index("s")

    def fill_ones(i, carry):
        ones_v[pl.ds(i * 16, 16)] = jnp.full((16,), 1.0, jnp.float32)
        return carry

    lax.fori_loop(0, 8, fill_ones, 0)

    def fill_zeros(i, carry):
        zeros_v[pl.ds(i * 16, 16)] = jnp.zeros((16,), jnp.float32)
        return carry

    lax.fori_loop(0, 40, fill_zeros, 0)

    pltpu.sync_copy(zeros_v, mask_hbm.at[pl.ds(c * NP + s * 640, 640)])
    plsc.subcore_barrier()

    mpt = M // NS

    def chunk(k, carry):
        off = c * M + s * mpt + k * 128
        pltpu.sync_copy(posflat_hbm.at[pl.ds(off, 128)], idx_v)

        def addoff(j, cc):
            fidx_v[pl.ds(j * 16, 16)] = idx_v[pl.ds(j * 16, 16)] + c * NP
            return cc

        lax.fori_loop(0, 8, addoff, 0)
        pltpu.sync_copy(ones_v, mask_hbm.at[fidx_v])
        return carry

    lax.fori_loop(0, mpt // 128, chunk, 0)


ECH = 128
NCHUNK = E // ECH


@functools.partial(
    pl.kernel,
    out_type=jax.ShapeDtypeStruct((E, F), jnp.float32),
    mesh=_mesh,
    scratch_types=[
        pltpu.VMEM((ECH,), jnp.int32),
        pltpu.VMEM((ECH, F), jnp.float32),
        pltpu.SemaphoreType.DMA,
    ],
)
def _msgs_kernel(xt_hbm, src_hbm, msgs_hbm, idx_v, rows_v, sem):
    c = lax.axis_index("c")
    s = lax.axis_index("s")
    wid = c * NS + s

    def ch(i, carry):
        k = wid + i * (NC * NS)

        @pl.when(k < NCHUNK)
        def _():
            r0 = k * ECH
            pltpu.sync_copy(src_hbm.at[pl.ds(r0, ECH)], idx_v)
            pltpu.async_copy(xt_hbm.at[idx_v], rows_v, sem).wait()
            pltpu.sync_copy(rows_v, msgs_hbm.at[pl.ds(r0, ECH)])

        return carry

    lax.fori_loop(0, (NCHUNK + NC * NS - 1) // (NC * NS), ch, 0)


@functools.partial(
    pl.kernel,
    out_type=(
        jax.ShapeDtypeStruct((4 * M, F), jnp.float32),
        jax.ShapeDtypeStruct((4 * M,), jnp.float32),
    ),
    mesh=_mesh,
    scratch_types=[
        pltpu.VMEM((128,), jnp.int32),
        pltpu.VMEM((128, F), jnp.float32),
        pltpu.VMEM((128,), jnp.float32),
        pltpu.SemaphoreType.DMA,
    ],
)
def _gather_kernel(table_hbm, pf_hbm, mf_hbm, mask_hbm, g_hbm, mv_hbm,
                   idx_v, rows_v, mrow_v, sem):
    c = lax.axis_index("c")
    s = lax.axis_index("s")
    wid = c * NS + s
    rpw = (4 * M) // (NC * NS)

    def ch(k, carry):
        r0 = wid * rpw + k * 128
        pltpu.sync_copy(pf_hbm.at[pl.ds(r0, 128)], idx_v)
        pltpu.async_copy(table_hbm.at[idx_v], rows_v, sem).wait()
        pltpu.sync_copy(rows_v, g_hbm.at[pl.ds(r0, 128)])
        pltpu.sync_copy(mf_hbm.at[pl.ds(r0, 128)], idx_v)
        pltpu.async_copy(mask_hbm.at[idx_v], mrow_v, sem).wait()
        pltpu.sync_copy(mrow_v, mv_hbm.at[pl.ds(r0, 128)])
        return carry

    lax.fori_loop(0, rpw // 128, ch, 0)


EB = 4000
NSTEP = E // EB


def _segsum_body(dst_smem, msgs_ref, agg_ref, deg_ref, acc_ref, dacc_s):
    i = pl.program_id(0)

    @pl.when(i == 0)
    def _():
        acc_ref[...] = jnp.zeros((N, F), jnp.float32)

        def zd(n, carry):
            dacc_s[0, n] = 0.0
            return carry

        lax.fori_loop(0, N, zd, 0)

    def edge(e, carry):
        d = dst_smem[0, 0, e]
        acc_ref[pl.ds(d, 1), :] = (
            acc_ref[pl.ds(d, 1), :] + msgs_ref[pl.ds(e, 1), :])
        dacc_s[0, d] = dacc_s[0, d] + 1.0
        return carry

    lax.fori_loop(0, EB, edge, 0)

    @pl.when(i == NSTEP - 1)
    def _():
        agg_ref[...] = acc_ref[...]

        def wd(n, carry):
            deg_ref[0, n] = dacc_s[0, n]
            return carry

        lax.fori_loop(0, N, wd, 0)


def _segsum_call(msgs, dstb):
    return pl.pallas_call(
        _segsum_body,
        grid=(NSTEP,),
        in_specs=[
            pl.BlockSpec((1, 1, EB), lambda i: (i, 0, 0),
                         memory_space=pltpu.SMEM),
            pl.BlockSpec((EB, F), lambda i: (i, 0)),
        ],
        out_specs=(
            pl.BlockSpec((N, F), lambda i: (0, 0)),
            pl.BlockSpec((1, N), lambda i: (0, 0),
                         memory_space=pltpu.SMEM),
        ),
        out_shape=(
            jax.ShapeDtypeStruct((N, F), jnp.float32),
            jax.ShapeDtypeStruct((1, N), jnp.float32),
        ),
        scratch_shapes=[
            pltpu.VMEM((N, F), jnp.float32),
            pltpu.SMEM((1, N), jnp.float32),
        ],
    )(dstb, msgs)


BLK = 1000


def _layer0_body(x_ref, w0_ref, b0_ref, w1_ref, b1_ref, m0_ref, m1_ref,
                 o0_ref, o1_ref):
    xx = x_ref[...]
    y0 = jnp.dot(xx, w0_ref[...], preferred_element_type=jnp.float32) + b0_ref[...]
    y1 = jnp.dot(xx, w1_ref[...], preferred_element_type=jnp.float32) + b1_ref[...]
    d = y1 - y0
    o0_ref[...] = y0 + m0_ref[...] * d
    o1_ref[...] = y0 + m1_ref[...] * d


def _layer0_call(x, w0, b0, w1, b1, m0, m1):
    grid = (N // BLK,)
    row = pl.BlockSpec((BLK, F), lambda i: (i, 0))
    full = pl.BlockSpec((F, F), lambda i: (0, 0))
    bias = pl.BlockSpec((1, F), lambda i: (0, 0))
    mspec = pl.BlockSpec((BLK, 1), lambda i: (i, 0))
    return pl.pallas_call(
        _layer0_body,
        grid=grid,
        in_specs=[row, full, bias, full, bias, mspec, mspec],
        out_specs=(row, row),
        out_shape=(
            jax.ShapeDtypeStruct((N, F), jnp.float32),
            jax.ShapeDtypeStruct((N, F), jnp.float32),
        ),
    )(x, w0, b0, w1, b1, m0, m1)


def _conv_body(agg_ref, deg_ref, wc_ref, bc_ref, o_ref):
    a = agg_ref[...] / jnp.maximum(deg_ref[...], 1.0)
    y = jnp.dot(a, wc_ref[...], preferred_element_type=jnp.float32) + bc_ref[...]
    o_ref[...] = jnp.maximum(y, 0.0)


def _conv_call(agg, deg, wc, bc):
    grid = (N // BLK,)
    row = pl.BlockSpec((BLK, F), lambda i: (i, 0))
    full = pl.BlockSpec((F, F), lambda i: (0, 0))
    bias = pl.BlockSpec((1, F), lambda i: (0, 0))
    dspec = pl.BlockSpec((BLK, 1), lambda i: (i, 0))
    return pl.pallas_call(
        _conv_body,
        grid=grid,
        in_specs=[row, dspec, full, bias],
        out_specs=row,
        out_shape=jax.ShapeDtypeStruct((N, F), jnp.float32),
    )(agg, deg, wc, bc)


def _final_body(g_ref, m_ref, w0_ref, b0_ref, w1_ref, b1_ref, o_ref):
    g = g_ref[...]
    y0 = jnp.dot(g, w0_ref[...], preferred_element_type=jnp.float32) + b0_ref[...]
    y1 = jnp.dot(g, w1_ref[...], preferred_element_type=jnp.float32) + b1_ref[...]
    o_ref[...] = y0 + m_ref[...] * (y1 - y0)


def _final_call(g, m, w0, b0, w1, b1):
    R = 4 * M
    blk = 1024
    grid = (R // blk,)
    row = pl.BlockSpec((blk, F), lambda i: (i, 0))
    full = pl.BlockSpec((F, F), lambda i: (0, 0))
    bias = pl.BlockSpec((1, F), lambda i: (0, 0))
    mspec = pl.BlockSpec((blk, 1), lambda i: (i, 0))
    return pl.pallas_call(
        _final_body,
        grid=grid,
        in_specs=[row, mspec, full, bias, full, bias],
        out_specs=row,
        out_shape=jax.ShapeDtypeStruct((R, F), jnp.float32),
    )(g, m, w0, b0, w1, b1)


def kernel(x, edge_index, pos, Wc, bc, W0, b0, W1, b1):
    src = edge_index[0]
    dst = edge_index[1]
    dstb = dst.reshape(NSTEP, 1, EB)
    posflat = pos.T.reshape(-1)

    maskflat = _mask_kernel(posflat)
    m0 = maskflat[0:N].reshape(N, 1)
    m1 = maskflat[NP:NP + N].reshape(N, 1)

    x00, x01 = _layer0_call(x, W0[0], b0[0].reshape(1, F), W1[0],
                            b1[0].reshape(1, F), m0, m1)

    msgs0 = _msgs_kernel(x00, src)
    msgs1 = _msgs_kernel(x01, src)
    agg0, deg = _segsum_call(msgs0, dstb)
    agg1, _deg1 = _segsum_call(msgs1, dstb)

    degc = deg.reshape(N, 1)
    xb0 = _conv_call(agg0, degc, Wc[0], bc[0].reshape(1, F))
    xb1 = _conv_call(agg1, degc, Wc[0], bc[0].reshape(1, F))

    table = jnp.concatenate([xb0, xb1], axis=0)
    p0 = pos.reshape(-1)
    p1 = pos[:, ::-1].reshape(-1)
    pf = jnp.concatenate([p0, N + p1])
    mf = jnp.concatenate([p0, NP + p1])

    g, mv = _gather_kernel(table, pf, mf, maskflat)

    res = _final_call(g, mv.reshape(4 * M, 1), W0[1], b0[1].reshape(1, F),
                      W1[1], b1[1].reshape(1, F))
    return res.reshape(2, M, 2, F).transpose(1, 2, 0, 3)
